# Initial kernel scaffold; baseline (speedup 1.0000x reference)
import jax
import jax.numpy as jnp
from jax.experimental import pallas as pl


def kernel(inputs, pos, pe1, pe2):
    raise NotImplementedError("write your pallas kernel here")


# speedup vs baseline: 2.4155x; 2.4155x over previous
import functools

import jax
import jax.numpy as jnp
from jax import lax
from jax.experimental import pallas as pl
from jax.experimental.pallas import tpu as pltpu
from jax.experimental.pallas import tpu_sc as plsc

B, S, D = 32, 2048, 128
H = D // 2
N2 = B * S * 2
W = 128


def _sc_body(x_hbm, posf_hbm, pe12_hbm, out_hbm, idx_s):
    off = (lax.iota(jnp.int32, 16) & 1) * 48

    def body(x_vmem, pos_vmem, out_vmem):
        @pl.loop(0, W, step=16)
        def _(k):
            idx_s[0, pl.ds(k, 16)] = pos_vmem[0, pl.ds(k, 16)] + off

        pltpu.sync_copy(pe12_hbm.at[idx_s.at[0]], out_vmem)

        @pl.loop(0, W)
        def _(r):
            for c in range(H // 16):
                slc = (pl.ds(r, 1), pl.ds(c * 16, 16))
                out_vmem.at[*slc][...] += x_vmem.at[*slc][...]

    pltpu.emit_pipeline(
        body,
        grid=(N2 // W,),
        in_specs=[
            pl.BlockSpec((W, H), lambda i: (i, 0)),
            pl.BlockSpec((1, W), lambda i: (0, i)),
        ],
        out_specs=[pl.BlockSpec((W, H), lambda i: (i, 0))],
        core_axis_name=("c", "s"),
        dimension_semantics=(pltpu.PARALLEL,),
    )(x_hbm, posf_hbm, out_hbm)


def kernel(inputs, pos, pe1, pe2):
    x2 = inputs.reshape(N2, H)
    posf = pos.astype(jnp.int32).reshape(1, N2)
    pe12 = jnp.concatenate([pe1, pe2], axis=0)
    mesh = plsc.VectorSubcoreMesh(core_axis_name="c", subcore_axis_name="s")
    run = functools.partial(
        pl.kernel,
        out_type=jax.ShapeDtypeStruct((N2, H), jnp.float32),
        mesh=mesh,
        scratch_types=[pltpu.VMEM((1, W), jnp.int32)],
        compiler_params=pltpu.CompilerParams(use_tc_tiling_on_sc=False),
    )(_sc_body)
    out2 = run(x2, posf, pe12)
    return out2.reshape(B, S, D)


# speedup vs baseline: 2.6811x; 1.1100x over previous
import functools

import jax
import jax.numpy as jnp
from jax import lax
from jax.experimental import pallas as pl
from jax.experimental.pallas import tpu as pltpu
from jax.experimental.pallas import tpu_sc as plsc

B, S, D = 32, 2048, 128
H = D // 2
N2 = B * S * 2
W = 128


def _sc_body(x_hbm, posf_hbm, pe12_hbm, out_hbm, idx_s):
    off = (lax.iota(jnp.int32, 16) & 1) * 48

    def body(x_vmem, pos_vmem, out_vmem):
        @pl.loop(0, W, step=16)
        def _(k):
            idx_s[0, pl.ds(k, 16)] = pos_vmem[0, pl.ds(k, 16)] + off

        @pl.loop(0, W)
        def _(r):
            for c in range(H // 16):
                slc = (pl.ds(r, 1), pl.ds(c * 16, 16))
                out_vmem.at[*slc][...] = x_vmem.at[*slc][...]

        pltpu.sync_copy(pe12_hbm.at[idx_s.at[0]], out_vmem, add=True)

    pltpu.emit_pipeline(
        body,
        grid=(N2 // W,),
        in_specs=[
            pl.BlockSpec((W, H), lambda i: (i, 0)),
            pl.BlockSpec((1, W), lambda i: (0, i)),
        ],
        out_specs=[pl.BlockSpec((W, H), lambda i: (i, 0))],
        core_axis_name=("c", "s"),
        dimension_semantics=(pltpu.PARALLEL,),
    )(x_hbm, posf_hbm, out_hbm)


def kernel(inputs, pos, pe1, pe2):
    x2 = inputs.reshape(N2, H)
    posf = pos.astype(jnp.int32).reshape(1, N2)
    pe12 = jnp.concatenate([pe1, pe2], axis=0)
    mesh = plsc.VectorSubcoreMesh(core_axis_name="c", subcore_axis_name="s")
    run = functools.partial(
        pl.kernel,
        out_type=jax.ShapeDtypeStruct((N2, H), jnp.float32),
        mesh=mesh,
        scratch_types=[pltpu.VMEM((1, W), jnp.int32)],
        compiler_params=pltpu.CompilerParams(use_tc_tiling_on_sc=False),
    )(_sc_body)
    out2 = run(x2, posf, pe12)
    return out2.reshape(B, S, D)


# speedup vs baseline: 3.0194x; 1.1262x over previous
import functools

import jax
import jax.numpy as jnp
from jax import lax
from jax.experimental import pallas as pl
from jax.experimental.pallas import tpu as pltpu
from jax.experimental.pallas import tpu_sc as plsc

B, S, D = 32, 2048, 128
H = D // 2
N2 = B * S * 2
NWORK = 32
RPW = N2 // NWORK
GW = 128
CW = 512
GPW = CW // GW
NWIN = RPW // CW
NBUF = 3


def _sc_body(x_hbm, posf_hbm, pe12_hbm, out_hbm,
             pos_v, idx_v, b0, b1, b2,
             si0, si1, si2, sg0, sg1, sg2, so0, so1, so2):
    bufs = (b0, b1, b2)
    sin = (si0, si1, si2)
    sga = (sg0, sg1, sg2)
    sout = (so0, so1, so2)

    wid = lax.axis_index("s") * 2 + lax.axis_index("c")
    base = wid * RPW

    pltpu.sync_copy(posf_hbm.at[pl.ds(base, RPW)], pos_v)
    off = (lax.iota(jnp.int32, 16) & 1) * 48

    @pl.loop(0, RPW // GW)
    def _(j):
        for k in range(GW // 16):
            idx_v[j, pl.ds(k * 16, 16)] = pos_v[pl.ds(j * GW + k * 16, 16)] + off

    def issue_in(w):
        b = w % NBUF
        return pltpu.async_copy(
            x_hbm.at[pl.ds(base + w * CW, CW)], bufs[b], sin[b])

    ins = [issue_in(0), issue_in(1)]
    outs = [None] * NWIN
    for w in range(NWIN):
        b = w % NBUF
        ins[w].wait()
        gas = [
            pltpu.async_copy(
                pe12_hbm.at[idx_v.at[w * GPW + j]],
                bufs[b].at[pl.ds(j * GW, GW)],
                sga[b], add=True)
            for j in range(GPW)
        ]
        for g in gas:
            g.wait()
        outs[w] = pltpu.async_copy(
            bufs[b], out_hbm.at[pl.ds(base + w * CW, CW)], sout[b])
        if w + 2 < NWIN:
            if w >= 1:
                outs[w - 1].wait()
            ins.append(issue_in(w + 2))
    outs[NWIN - 2].wait()
    outs[NWIN - 1].wait()


def kernel(inputs, pos, pe1, pe2):
    x2 = inputs.reshape(N2, H)
    posf = pos.astype(jnp.int32).reshape(N2)
    pe12 = jnp.concatenate([pe1, pe2], axis=0)
    mesh = plsc.VectorSubcoreMesh(core_axis_name="c", subcore_axis_name="s")
    run = functools.partial(
        pl.kernel,
        out_type=jax.ShapeDtypeStruct((N2, H), jnp.float32),
        mesh=mesh,
        scratch_types=[
            pltpu.VMEM((RPW,), jnp.int32),
            pltpu.VMEM((RPW // GW, GW), jnp.int32),
            pltpu.VMEM((CW, H), jnp.float32),
            pltpu.VMEM((CW, H), jnp.float32),
            pltpu.VMEM((CW, H), jnp.float32),
        ] + [pltpu.SemaphoreType.DMA] * 9,
        compiler_params=pltpu.CompilerParams(use_tc_tiling_on_sc=False),
    )(_sc_body)
    out2 = run(x2, posf, pe12)
    return out2.reshape(B, S, D)


# speedup vs baseline: 5.9782x; 1.9800x over previous
import functools

import jax
import jax.numpy as jnp
from jax import lax
from jax.experimental import pallas as pl
from jax.experimental.pallas import tpu as pltpu
from jax.experimental.pallas import tpu_sc as plsc

B, S, D = 32, 2048, 128
H = D // 2
N2 = B * S * 2
NWORK = 32
RPW = N2 // NWORK
GW = 128
CW = 512
GPW = CW // GW
NWIN = RPW // CW
NBUF = 3


def _sc_body(x_hbm, posf_hbm, pe12_hbm, out_hbm,
             pe_sh, pos_v, idx_v, b0, b1, b2,
             si0, si1, si2, sg0, sg1, sg2, so0, so1, so2):
    bufs = (b0, b1, b2)
    sin = (si0, si1, si2)
    sga = (sg0, sg1, sg2)
    sout = (so0, so1, so2)

    wid = lax.axis_index("s") * 2 + lax.axis_index("c")
    base = wid * RPW

    @pl.when(lax.axis_index("s") == 0)
    def _():
        pltpu.sync_copy(pe12_hbm, pe_sh)

    plsc.subcore_barrier()

    pltpu.sync_copy(posf_hbm.at[pl.ds(base, RPW)], pos_v)
    off = (lax.iota(jnp.int32, 16) & 1) * 48

    @pl.loop(0, RPW // GW)
    def _(j):
        for k in range(GW // 16):
            idx_v[j, pl.ds(k * 16, 16)] = pos_v[pl.ds(j * GW + k * 16, 16)] + off

    def issue_in(w):
        b = w % NBUF
        return pltpu.async_copy(
            x_hbm.at[pl.ds(base + w * CW, CW)], bufs[b], sin[b])

    ins = [issue_in(0), issue_in(1)]
    outs = [None] * NWIN
    for w in range(NWIN):
        b = w % NBUF
        ins[w].wait()
        gas = [
            pltpu.async_copy(
                pe_sh.at[idx_v.at[w * GPW + j]],
                bufs[b].at[pl.ds(j * GW, GW)],
                sga[b], add=True)
            for j in range(GPW)
        ]
        for g in gas:
            g.wait()
        outs[w] = pltpu.async_copy(
            bufs[b], out_hbm.at[pl.ds(base + w * CW, CW)], sout[b])
        if w + 2 < NWIN:
            if w >= 1:
                outs[w - 1].wait()
            ins.append(issue_in(w + 2))
    outs[NWIN - 2].wait()
    outs[NWIN - 1].wait()


def kernel(inputs, pos, pe1, pe2):
    x2 = inputs.reshape(N2, H)
    posf = pos.astype(jnp.int32).reshape(N2)
    pe12 = jnp.concatenate([pe1, pe2], axis=0)
    mesh = plsc.VectorSubcoreMesh(core_axis_name="c", subcore_axis_name="s")
    run = functools.partial(
        pl.kernel,
        out_type=jax.ShapeDtypeStruct((N2, H), jnp.float32),
        mesh=mesh,
        scratch_types=[
            pltpu.VMEM_SHARED((304, H), jnp.float32),
            pltpu.VMEM((RPW,), jnp.int32),
            pltpu.VMEM((RPW // GW, GW), jnp.int32),
            pltpu.VMEM((CW, H), jnp.float32),
            pltpu.VMEM((CW, H), jnp.float32),
            pltpu.VMEM((CW, H), jnp.float32),
        ] + [pltpu.SemaphoreType.DMA] * 9,
        compiler_params=pltpu.CompilerParams(use_tc_tiling_on_sc=False),
    )(_sc_body)
    out2 = run(x2, posf, pe12)
    return out2.reshape(B, S, D)
